# baseline (speedup 1.0000x reference)
import functools

import jax
import jax.numpy as jnp
from jax import lax
from jax.experimental import pallas as pl
from jax.experimental.pallas import tpu as pltpu
from jax.experimental.pallas import tpu_sc as plsc

NUM_CORES = 2
NUM_SUBCORES = 16
NUM_WORKERS = NUM_CORES * NUM_SUBCORES


@functools.cache
def _make_kernel(b: int, l: int, dim: int):
    b_per_w = b // NUM_WORKERS
    tcols = b_per_w // 128
    assert l % 2 == 0 and b_per_w % 128 == 0 and dim % 8 == 0

    mesh = plsc.VectorSubcoreMesh(
        core_axis_name="c", subcore_axis_name="s",
        num_cores=NUM_CORES, num_subcores=NUM_SUBCORES)

    @functools.partial(
        pl.kernel,
        out_type=jax.ShapeDtypeStruct((l, dim // 8, b // 128, 8, 128),
                                      jnp.float32),
        mesh=mesh,
        scratch_types=[
            pltpu.VMEM((l, b_per_w), jnp.int32),
            pltpu.VMEM((2, b_per_w, dim), jnp.float32),
            pltpu.VMEM((2, dim // 8, tcols, 8, 128), jnp.float32),
            pltpu.SemaphoreType.DMA,
            pltpu.SemaphoreType.DMA,
            pltpu.SemaphoreType.DMA,
            pltpu.SemaphoreType.DMA,
        ],
        compiler_params=pltpu.CompilerParams(
            use_tc_tiling_on_sc=False, needs_layout_passes=False),
    )
    def emb(xt_hbm, table_hbm, out_hbm, xv, rows, tr5, gs0, gs1, ws0, ws1):
        wid = lax.axis_index("s") * NUM_CORES + lax.axis_index("c")
        b0 = wid * b_per_w
        tc0 = wid * tcols
        pltpu.sync_copy(xt_hbm.at[:, pl.ds(b0, b_per_w)], xv)

        gsems = (gs0, gs1)
        wsems = (ws0, ws1)

        def fire(pos, p):
            pltpu.async_copy(
                table_hbm.at[xv.at[pos]], rows.at[p], gsems[p])

        def wait_gather(p):
            pltpu.make_async_copy(
                table_hbm.at[xv.at[0]], rows.at[p], gsems[p]).wait()

        def start_wb(pos, p):
            pltpu.async_copy(
                tr5.at[p], out_hbm.at[pos, :, pl.ds(tc0, tcols)], wsems[p])

        def wait_wb(p):
            pltpu.make_async_copy(
                tr5.at[p], out_hbm.at[0, :, pl.ds(tc0, tcols)],
                wsems[p]).wait()

        row_bases = [
            jnp.arange(16, dtype=jnp.int32) + (tb * 128 + c0 * 16)
            for tb in range(tcols) for c0 in range(8)
        ]

        def transpose(p):
            src = rows.at[p]
            dst = tr5.at[p]

            def tbody(j, carry):
                a = j // 8
                rr = j % 8
                col = jnp.full((16,), j, dtype=jnp.int32)
                for tb in range(tcols):
                    for c0 in range(8):
                        vec = plsc.load_gather(
                            src, [row_bases[tb * 8 + c0], col])
                        dst[a, tb, rr, pl.ds(c0 * 16, 16)] = vec
                return carry

            lax.fori_loop(0, dim, tbody, 0)

        fire(0, 0)
        fire(1, 1)

        def body(j, carry):
            p0 = 2 * j
            wait_gather(0)

            @pl.when(j > 0)
            def _():
                wait_wb(0)

            transpose(0)
            start_wb(p0, 0)

            @pl.when(j < l // 2 - 1)
            def _():
                fire(p0 + 2, 0)

            wait_gather(1)

            @pl.when(j > 0)
            def _():
                wait_wb(1)

            transpose(1)
            start_wb(p0 + 1, 1)

            @pl.when(j < l // 2 - 1)
            def _():
                fire(p0 + 3, 1)

            return carry

        lax.fori_loop(0, l // 2, body, 0)
        wait_wb(0)
        wait_wb(1)

    return emb


def kernel(x, weight):
    b, l = x.shape
    _, dim = weight.shape
    xt = x.T.astype(jnp.int32)
    out5 = _make_kernel(b, l, dim)(xt, weight)
    return out5.transpose(2, 4, 0, 1, 3).reshape(b, l, dim)


# speedup vs baseline: 1.1056x; 1.1056x over previous
import functools

import jax
import jax.numpy as jnp
from jax import lax
from jax.experimental import pallas as pl
from jax.experimental.pallas import tpu as pltpu
from jax.experimental.pallas import tpu_sc as plsc

NUM_CORES = 2
NUM_SUBCORES = 16
NUM_WORKERS = NUM_CORES * NUM_SUBCORES


@functools.cache
def _make_kernel(b: int, l: int, dim: int):
    b_per_w = b // NUM_WORKERS
    tcols = b_per_w // 128
    assert l % 2 == 0 and b_per_w % 128 == 0 and dim % 8 == 0

    mesh = plsc.VectorSubcoreMesh(
        core_axis_name="c", subcore_axis_name="s",
        num_cores=NUM_CORES, num_subcores=NUM_SUBCORES)

    @functools.partial(
        pl.kernel,
        out_type=jax.ShapeDtypeStruct((l, dim // 8, b // 128, 8, 128),
                                      jnp.float32),
        mesh=mesh,
        scratch_types=[
            pltpu.VMEM((l, b_per_w), jnp.int32),
            pltpu.VMEM((2, b_per_w, dim), jnp.float32),
            pltpu.VMEM((2, dim // 8, tcols, 8, 128), jnp.float32),
            pltpu.SemaphoreType.DMA,
            pltpu.SemaphoreType.DMA,
            pltpu.SemaphoreType.DMA,
            pltpu.SemaphoreType.DMA,
        ],
        compiler_params=pltpu.CompilerParams(
            use_tc_tiling_on_sc=False, needs_layout_passes=False),
    )
    def emb(xt_hbm, table_hbm, out_hbm, xv, rows, tr5, gs0, gs1, ws0, ws1):
        wid = lax.axis_index("s") * NUM_CORES + lax.axis_index("c")
        b0 = wid * b_per_w
        tc0 = wid * tcols
        pltpu.sync_copy(xt_hbm.at[:, pl.ds(b0, b_per_w)], xv)

        gsems = (gs0, gs1)
        wsems = (ws0, ws1)

        def fire(pos, p):
            pltpu.async_copy(
                table_hbm.at[xv.at[pos]], rows.at[p], gsems[p])

        def wait_gather(p):
            pltpu.make_async_copy(
                table_hbm.at[xv.at[0]], rows.at[p], gsems[p]).wait()

        def start_wb(pos, p):
            pltpu.async_copy(
                tr5.at[p], out_hbm.at[pos, :, pl.ds(tc0, tcols)], wsems[p])

        def wait_wb(p):
            pltpu.make_async_copy(
                tr5.at[p], out_hbm.at[0, :, pl.ds(tc0, tcols)],
                wsems[p]).wait()

        row_bases = [
            jnp.arange(16, dtype=jnp.int32) + (tb * 128 + c0 * 16)
            for tb in range(tcols) for c0 in range(8)
        ]

        def transpose(p):
            src = rows.at[p]
            dst = tr5.at[p]

            def tbody(j, carry):
                a = j // 8
                rr = j % 8
                col = jnp.full((16,), j, dtype=jnp.int32)
                vecs = [
                    plsc.load_gather(src, [row_bases[g], col])
                    for g in range(tcols * 8)
                ]
                for tb in range(tcols):
                    for c0 in range(8):
                        dst[a, tb, rr, pl.ds(c0 * 16, 16)] = vecs[tb * 8 + c0]
                return carry

            lax.fori_loop(0, dim, tbody, 0)

        fire(0, 0)
        fire(1, 1)

        def body(j, carry):
            p0 = 2 * j
            wait_gather(0)

            @pl.when(j > 0)
            def _():
                wait_wb(0)

            transpose(0)
            start_wb(p0, 0)

            @pl.when(j < l // 2 - 1)
            def _():
                fire(p0 + 2, 0)

            wait_gather(1)

            @pl.when(j > 0)
            def _():
                wait_wb(1)

            transpose(1)
            start_wb(p0 + 1, 1)

            @pl.when(j < l // 2 - 1)
            def _():
                fire(p0 + 3, 1)

            return carry

        lax.fori_loop(0, l // 2, body, 0)
        wait_wb(0)
        wait_wb(1)

    return emb


def kernel(x, weight):
    b, l = x.shape
    _, dim = weight.shape
    xt = x.T.astype(jnp.int32)
    out5 = _make_kernel(b, l, dim)(xt, weight)
    return out5.transpose(2, 4, 0, 1, 3).reshape(b, l, dim)
